# Initial kernel scaffold; baseline (speedup 1.0000x reference)
import jax
import jax.numpy as jnp
from jax.experimental import pallas as pl


def kernel(x, edge_weight, W_pre1, b_pre1, W_pre2, b_pre2, W_enc, b_enc, W_post1, b_post1, W_post2, b_post2, edge_index, batch, y):
    raise NotImplementedError("write your pallas kernel here")


# speedup vs baseline: 5.1310x; 5.1310x over previous
import functools

import jax
import jax.numpy as jnp
from jax import lax
from jax.experimental import pallas as pl
from jax.experimental.pallas import tpu as pltpu
from jax.experimental.pallas import tpu_sc as plsc

_NC = 2
_NS = 16
_L = 16
_NW = _NC * _NS

_CH = 128


def _pre_mlp(x, W1, b1, W2, b2):
  n, d = x.shape
  h = W1.shape[1]
  blk = 1000
  grid = n // blk

  def body(x_ref, w1_ref, b1_ref, w2_ref, b2_ref, o_ref):
    a = jnp.dot(x_ref[...], w1_ref[...], preferred_element_type=jnp.float32)
    a = jnp.maximum(a + b1_ref[...], 0.0)
    o_ref[...] = (
        jnp.dot(a, w2_ref[...], preferred_element_type=jnp.float32)
        + b2_ref[...]
    )

  return pl.pallas_call(
      body,
      grid=(grid,),
      in_specs=[
          pl.BlockSpec((blk, d), lambda i: (i, 0)),
          pl.BlockSpec((d, h), lambda i: (0, 0)),
          pl.BlockSpec((1, h), lambda i: (0, 0)),
          pl.BlockSpec((h, h), lambda i: (0, 0)),
          pl.BlockSpec((1, h), lambda i: (0, 0)),
      ],
      out_specs=pl.BlockSpec((blk, h), lambda i: (i, 0)),
      out_shape=jax.ShapeDtypeStruct((n, h), jnp.float32),
  )(x, W1, b1.reshape(1, -1), W2, b2.reshape(1, -1))


def _message_passing(h, src, dst, w):
  n, dim = h.shape
  e = src.shape[0]
  n_chunks = e // _CH
  chunks_per_w = pl.cdiv(n_chunks, _NW)
  rpt = (n // _NS) // 8 * 8
  rem = n - rpt * _NS

  mesh = plsc.VectorSubcoreMesh(core_axis_name="c", subcore_axis_name="s")

  @functools.partial(
      pl.kernel,
      out_type=jax.ShapeDtypeStruct((_NC, n, dim), jnp.float32),
      mesh=mesh,
      scratch_types=[
          pltpu.VMEM((1, _CH), jnp.int32),
          pltpu.VMEM((1, _CH), jnp.int32),
          pltpu.VMEM((_CH,), jnp.float32),
          pltpu.VMEM((1, _CH, 128), jnp.float32),
          pltpu.VMEM_SHARED((10000, 128), jnp.float32),
          pltpu.SemaphoreType.DMA,
      ],
  )
  def body(h_hbm, src_hbm, dst_hbm, w_hbm, out_hbm,
           src_v, dst_v, w_v, rows_v, acc_sh, sem):
    c = lax.axis_index("c")
    s = lax.axis_index("s")
    wid = s * _NC + c

    def zrow(i, carry):
      for g in range(dim // _L):
        rows_v[0, i, pl.ds(g * _L, _L)] = jnp.zeros((_L,), jnp.float32)
      return carry

    lax.fori_loop(0, _CH, zrow, 0)
    nfull, tail = divmod(rpt, _CH)
    for k in range(nfull):
      pltpu.sync_copy(
          rows_v.at[0],
          acc_sh.at[pl.ds(s * rpt + k * _CH, _CH)],
      )
    if tail:
      pltpu.sync_copy(
          rows_v.at[0, pl.ds(0, tail)],
          acc_sh.at[pl.ds(s * rpt + nfull * _CH, tail)],
      )
    if rem:
      @pl.when(s == 0)
      def _():
        pltpu.sync_copy(
            rows_v.at[0, pl.ds(0, rem)],
            acc_sh.at[pl.ds(_NS * rpt, rem)],
        )
    plsc.subcore_barrier()

    def chunk_body(i, carry):
      cid = wid * chunks_per_w + i

      @pl.when(cid < n_chunks)
      def _():
        base = cid * _CH
        pltpu.sync_copy(src_hbm.at[pl.ds(base, _CH)], src_v.at[0])
        pltpu.sync_copy(dst_hbm.at[pl.ds(base, _CH)], dst_v.at[0])
        pltpu.sync_copy(w_hbm.at[pl.ds(base, _CH)], w_v)
        pltpu.async_copy(h_hbm.at[src_v.at[0]], rows_v.at[0], sem).wait()

        def grp_body(g, ecarry):
          wgrp = w_v[pl.ds(g * _L, _L)]
          for j in range(_L):
            ei = g * _L + j
            wb = lax.gather(
                wgrp,
                jnp.full((_L, 1), j, jnp.int32),
                lax.GatherDimensionNumbers(
                    offset_dims=(),
                    collapsed_slice_dims=(0,),
                    start_index_map=(0,),
                ),
                (1,),
                mode=lax.GatherScatterMode.PROMISE_IN_BOUNDS,
            )
            for gg in range(dim // _L):
              rows_v[0, ei, pl.ds(gg * _L, _L)] = (
                  rows_v[0, ei, pl.ds(gg * _L, _L)] * wb
              )
          return ecarry

        lax.fori_loop(0, _CH // _L, grp_body, 0)
        pltpu.sync_copy(rows_v.at[0], acc_sh.at[dst_v.at[0]], add=True)

      return carry

    lax.fori_loop(0, chunks_per_w, chunk_body, 0)
    plsc.subcore_barrier()

    pltpu.sync_copy(
        acc_sh.at[pl.ds(s * rpt, rpt)],
        out_hbm.at[c, pl.ds(s * rpt, rpt)],
    )
    if rem:
      @pl.when(s == 0)
      def _():
        pltpu.sync_copy(
            acc_sh.at[pl.ds(_NS * rpt, rem)],
            out_hbm.at[c, pl.ds(_NS * rpt, rem)],
        )

  return body(h, src, dst, w)


def _readout(parts, batch3d, W_enc, b_enc, W1, b1, W2, b2):
  n = parts.shape[1]
  dim = parts.shape[2]
  nb = batch3d.shape[0]
  blk = n // nb
  ngraphs = 64
  out_d = W2.shape[1]

  def body(p0_ref, p1_ref, b_ref, we_ref, be_ref, w1_ref, b1_ref, w2_ref,
           b2_ref, o_ref, g_acc):
    i = pl.program_id(0)
    agg = p0_ref[0] + p1_ref[0]
    h2 = jnp.dot(agg, we_ref[...], preferred_element_type=jnp.float32)
    h2 = jnp.maximum(h2 + be_ref[...], 0.0)
    bvec = b_ref[0, 0, :]
    onehot = (
        bvec[None, :]
        == lax.broadcasted_iota(jnp.int32, (ngraphs, blk), 0)
    ).astype(jnp.float32)

    @pl.when(i == 0)
    def _():
      g_acc[...] = jnp.zeros_like(g_acc)

    g_acc[...] += jnp.dot(onehot, h2, preferred_element_type=jnp.float32)

    @pl.when(i == nb - 1)
    def _():
      g = g_acc[...]
      o = jnp.dot(g, w1_ref[...], preferred_element_type=jnp.float32)
      o = jnp.maximum(o + b1_ref[...], 0.0)
      o_ref[...] = (
          jnp.dot(o, w2_ref[...], preferred_element_type=jnp.float32)
          + b2_ref[...]
      )

  return pl.pallas_call(
      body,
      grid=(nb,),
      in_specs=[
          pl.BlockSpec((1, blk, dim), lambda i: (0, i, 0)),
          pl.BlockSpec((1, blk, dim), lambda i: (1, i, 0)),
          pl.BlockSpec((1, 1, blk), lambda i: (i, 0, 0)),
          pl.BlockSpec((dim, dim), lambda i: (0, 0)),
          pl.BlockSpec((1, dim), lambda i: (0, 0)),
          pl.BlockSpec((dim, dim), lambda i: (0, 0)),
          pl.BlockSpec((1, dim), lambda i: (0, 0)),
          pl.BlockSpec((dim, out_d), lambda i: (0, 0)),
          pl.BlockSpec((1, out_d), lambda i: (0, 0)),
      ],
      out_specs=pl.BlockSpec((ngraphs, out_d), lambda i: (0, 0)),
      out_shape=jax.ShapeDtypeStruct((ngraphs, out_d), jnp.float32),
      scratch_shapes=[pltpu.VMEM((ngraphs, dim), jnp.float32)],
  )(parts, parts, batch3d, W_enc, b_enc.reshape(1, -1), W1,
    b1.reshape(1, -1), W2, b2.reshape(1, -1))


def kernel(x, edge_weight, W_pre1, b_pre1, W_pre2, b_pre2, W_enc, b_enc,
           W_post1, b_post1, W_post2, b_post2, edge_index, batch, y):
  h = _pre_mlp(x, W_pre1, b_pre1, W_pre2, b_pre2)
  src = edge_index[0].astype(jnp.int32)
  dst = edge_index[1].astype(jnp.int32)
  parts = _message_passing(h, src, dst, edge_weight)
  n = x.shape[0]
  nb = 10
  batch3d = batch.astype(jnp.int32).reshape(nb, 1, n // nb)
  out = _readout(parts, batch3d, W_enc, b_enc, W_post1, b_post1,
                 W_post2, b_post2)
  return (out, y)


# speedup vs baseline: 9.7533x; 1.9009x over previous
import functools

import jax
import jax.numpy as jnp
from jax import lax
from jax.experimental import pallas as pl
from jax.experimental.pallas import tpu as pltpu
from jax.experimental.pallas import tpu_sc as plsc

_NC = 2
_NS = 16
_L = 16
_NW = _NC * _NS

_CH = 128


def _pre_mlp(x, W1, b1, W2, b2):
  n, d = x.shape
  h = W1.shape[1]
  blk = 1000
  grid = n // blk

  def body(x_ref, w1_ref, b1_ref, w2_ref, b2_ref, o_ref):
    a = jnp.dot(x_ref[...], w1_ref[...], preferred_element_type=jnp.float32)
    a = jnp.maximum(a + b1_ref[...], 0.0)
    o_ref[...] = (
        jnp.dot(a, w2_ref[...], preferred_element_type=jnp.float32)
        + b2_ref[...]
    )

  return pl.pallas_call(
      body,
      grid=(grid,),
      in_specs=[
          pl.BlockSpec((blk, d), lambda i: (i, 0)),
          pl.BlockSpec((d, h), lambda i: (0, 0)),
          pl.BlockSpec((1, h), lambda i: (0, 0)),
          pl.BlockSpec((h, h), lambda i: (0, 0)),
          pl.BlockSpec((1, h), lambda i: (0, 0)),
      ],
      out_specs=pl.BlockSpec((blk, h), lambda i: (i, 0)),
      out_shape=jax.ShapeDtypeStruct((n, h), jnp.float32),
  )(x, W1, b1.reshape(1, -1), W2, b2.reshape(1, -1))


def _message_passing(h, edges, w):
  n, dim = h.shape
  e_pad = edges.shape[1]
  cpw = e_pad // (_NW * _CH)
  nbuf = 3
  rpt = (n // _NS) // 8 * 8
  rem = n - rpt * _NS

  mesh = plsc.VectorSubcoreMesh(core_axis_name="c", subcore_axis_name="s")

  @functools.partial(
      pl.kernel,
      out_type=jax.ShapeDtypeStruct((_NC, n, dim), jnp.float32),
      mesh=mesh,
      scratch_types=[
          pltpu.VMEM((nbuf, 2, _CH), jnp.int32),
          pltpu.VMEM((nbuf, _CH), jnp.float32),
          pltpu.VMEM((nbuf, _CH, 128), jnp.float32),
          pltpu.VMEM_SHARED((10000, 128), jnp.float32),
          pltpu.SemaphoreType.DMA,
          pltpu.SemaphoreType.DMA,
          pltpu.SemaphoreType.DMA,
          pltpu.SemaphoreType.DMA,
          pltpu.SemaphoreType.DMA,
          pltpu.SemaphoreType.DMA,
      ],
  )
  def body(h_hbm, edges_hbm, w_hbm, out_hbm,
           idx_v, w_v, rows_v, acc_sh, g0, g1, g2, s0, s1, s2):
    gsem = (g0, g1, g2)
    ssem = (s0, s1, s2)
    c = lax.axis_index("c")
    s = lax.axis_index("s")
    wid = s * _NC + c
    first = wid * cpw

    def zrow(i, carry):
      for g in range(dim // _L):
        rows_v[0, i, pl.ds(g * _L, _L)] = jnp.zeros((_L,), jnp.float32)
      return carry

    lax.fori_loop(0, _CH, zrow, 0)
    nfull, tail = divmod(rpt, _CH)
    for k in range(nfull):
      pltpu.sync_copy(
          rows_v.at[0],
          acc_sh.at[pl.ds(s * rpt + k * _CH, _CH)],
      )
    if tail:
      pltpu.sync_copy(
          rows_v.at[0, pl.ds(0, tail)],
          acc_sh.at[pl.ds(s * rpt + nfull * _CH, tail)],
      )
    if rem:
      @pl.when(s == 0)
      def _():
        pltpu.sync_copy(
            rows_v.at[0, pl.ds(0, rem)],
            acc_sh.at[pl.ds(_NS * rpt, rem)],
        )
    plsc.subcore_barrier()

    def issue(b, cid):
      base = cid * _CH
      pltpu.sync_copy(edges_hbm.at[:, pl.ds(base, _CH)], idx_v.at[b])
      pltpu.sync_copy(w_hbm.at[pl.ds(base, _CH)], w_v.at[b])
      pltpu.async_copy(h_hbm.at[idx_v.at[b, 0]], rows_v.at[b], gsem[b])

    def scale(b):
      def grp_body(g, ecarry):
        wgrp = w_v[b, pl.ds(g * _L, _L)]
        for j in range(_L):
          ei = g * _L + j
          wb = lax.gather(
              wgrp,
              jnp.full((_L, 1), j, jnp.int32),
              lax.GatherDimensionNumbers(
                  offset_dims=(),
                  collapsed_slice_dims=(0,),
                  start_index_map=(0,),
              ),
              (1,),
              mode=lax.GatherScatterMode.PROMISE_IN_BOUNDS,
          )
          for gg in range(dim // _L):
            rows_v[b, ei, pl.ds(gg * _L, _L)] = (
                rows_v[b, ei, pl.ds(gg * _L, _L)] * wb
            )
        return ecarry

      lax.fori_loop(0, _CH // _L, grp_body, 0)

    issue(0, first)

    def step(t, carry):
      for k in range(nbuf):
        b = k
        b1 = (k + 1) % nbuf
        i = t * nbuf + k
        cid = first + i

        @pl.when(i >= 2)
        def _():
          pltpu.make_async_copy(
              rows_v.at[b1], acc_sh.at[idx_v.at[b1, 1]], ssem[b1]
          ).wait()

        @pl.when(i + 1 < cpw)
        def _():
          issue(b1, cid + 1)

        pltpu.make_async_copy(
            h_hbm.at[idx_v.at[b, 0]], rows_v.at[b], gsem[b]
        ).wait()
        scale(b)
        pltpu.async_copy(
            rows_v.at[b], acc_sh.at[idx_v.at[b, 1]], ssem[b], add=True
        )
      return carry

    lax.fori_loop(0, cpw // nbuf, step, 0)
    for b in ((cpw - 2) % nbuf, (cpw - 1) % nbuf):
      pltpu.make_async_copy(
          rows_v.at[b], acc_sh.at[idx_v.at[b, 1]], ssem[b]
      ).wait()
    plsc.subcore_barrier()

    pltpu.sync_copy(
        acc_sh.at[pl.ds(s * rpt, rpt)],
        out_hbm.at[c, pl.ds(s * rpt, rpt)],
    )
    if rem:
      @pl.when(s == 0)
      def _():
        pltpu.sync_copy(
            acc_sh.at[pl.ds(_NS * rpt, rem)],
            out_hbm.at[c, pl.ds(_NS * rpt, rem)],
        )

  return body(h, edges, w)


def _readout(parts, batch3d, W_enc, b_enc, W1, b1, W2, b2):
  n = parts.shape[1]
  dim = parts.shape[2]
  nb = batch3d.shape[0]
  blk = n // nb
  ngraphs = 64
  out_d = W2.shape[1]

  def body(p0_ref, p1_ref, b_ref, we_ref, be_ref, w1_ref, b1_ref, w2_ref,
           b2_ref, o_ref, g_acc):
    i = pl.program_id(0)
    agg = p0_ref[0] + p1_ref[0]
    h2 = jnp.dot(agg, we_ref[...], preferred_element_type=jnp.float32)
    h2 = jnp.maximum(h2 + be_ref[...], 0.0)
    bvec = b_ref[0, 0, :]
    onehot = (
        bvec[None, :]
        == lax.broadcasted_iota(jnp.int32, (ngraphs, blk), 0)
    ).astype(jnp.float32)

    @pl.when(i == 0)
    def _():
      g_acc[...] = jnp.zeros_like(g_acc)

    g_acc[...] += jnp.dot(onehot, h2, preferred_element_type=jnp.float32)

    @pl.when(i == nb - 1)
    def _():
      g = g_acc[...]
      o = jnp.dot(g, w1_ref[...], preferred_element_type=jnp.float32)
      o = jnp.maximum(o + b1_ref[...], 0.0)
      o_ref[...] = (
          jnp.dot(o, w2_ref[...], preferred_element_type=jnp.float32)
          + b2_ref[...]
      )

  return pl.pallas_call(
      body,
      grid=(nb,),
      in_specs=[
          pl.BlockSpec((1, blk, dim), lambda i: (0, i, 0)),
          pl.BlockSpec((1, blk, dim), lambda i: (1, i, 0)),
          pl.BlockSpec((1, 1, blk), lambda i: (i, 0, 0)),
          pl.BlockSpec((dim, dim), lambda i: (0, 0)),
          pl.BlockSpec((1, dim), lambda i: (0, 0)),
          pl.BlockSpec((dim, dim), lambda i: (0, 0)),
          pl.BlockSpec((1, dim), lambda i: (0, 0)),
          pl.BlockSpec((dim, out_d), lambda i: (0, 0)),
          pl.BlockSpec((1, out_d), lambda i: (0, 0)),
      ],
      out_specs=pl.BlockSpec((ngraphs, out_d), lambda i: (0, 0)),
      out_shape=jax.ShapeDtypeStruct((ngraphs, out_d), jnp.float32),
      scratch_shapes=[pltpu.VMEM((ngraphs, dim), jnp.float32)],
  )(parts, parts, batch3d, W_enc, b_enc.reshape(1, -1), W1,
    b1.reshape(1, -1), W2, b2.reshape(1, -1))


def kernel(x, edge_weight, W_pre1, b_pre1, W_pre2, b_pre2, W_enc, b_enc,
           W_post1, b_post1, W_post2, b_post2, edge_index, batch, y):
  h = _pre_mlp(x, W_pre1, b_pre1, W_pre2, b_pre2)
  n = x.shape[0]
  e = edge_weight.shape[0]
  cpw = -(-e // (_NW * _CH))
  cpw = -(-cpw // 3) * 3
  pad = cpw * _NW * _CH - e
  ei32 = edge_index.astype(jnp.int32)
  if pad:
    ar = jnp.arange(pad, dtype=jnp.int32) % n
    edges = jnp.concatenate([ei32, jnp.stack([ar, ar])], axis=1)
    w_all = jnp.concatenate(
        [edge_weight, jnp.zeros((pad,), edge_weight.dtype)]
    )
  else:
    edges, w_all = ei32, edge_weight
  parts = _message_passing(h, edges, w_all)
  nb = 10
  batch3d = batch.astype(jnp.int32).reshape(nb, 1, n // nb)
  out = _readout(parts, batch3d, W_enc, b_enc, W_post1, b_post1,
                 W_post2, b_post2)
  return (out, y)


# speedup vs baseline: 11.5937x; 1.1887x over previous
import functools

import jax
import jax.numpy as jnp
from jax import lax
from jax.experimental import pallas as pl
from jax.experimental.pallas import tpu as pltpu
from jax.experimental.pallas import tpu_sc as plsc

_NC = 2
_NS = 16
_L = 16
_NW = _NC * _NS

_CH = 128


def _pre_mlp(x, W1, b1, W2, b2):
  n, d = x.shape
  h = W1.shape[1]
  blk = 1000
  grid = n // blk

  def body(x_ref, w1_ref, b1_ref, w2_ref, b2_ref, o_ref):
    a = jnp.dot(x_ref[...], w1_ref[...], preferred_element_type=jnp.float32)
    a = jnp.maximum(a + b1_ref[...], 0.0)
    o_ref[...] = (
        jnp.dot(a, w2_ref[...], preferred_element_type=jnp.float32)
        + b2_ref[...]
    )

  return pl.pallas_call(
      body,
      grid=(grid,),
      in_specs=[
          pl.BlockSpec((blk, d), lambda i: (i, 0)),
          pl.BlockSpec((d, h), lambda i: (0, 0)),
          pl.BlockSpec((1, h), lambda i: (0, 0)),
          pl.BlockSpec((h, h), lambda i: (0, 0)),
          pl.BlockSpec((1, h), lambda i: (0, 0)),
      ],
      out_specs=pl.BlockSpec((blk, h), lambda i: (i, 0)),
      out_shape=jax.ShapeDtypeStruct((n, h), jnp.float32),
  )(x, W1, b1.reshape(1, -1), W2, b2.reshape(1, -1))


def _message_passing(h, src2, dst3, wch):
  n, dim = h.shape
  nch = src2.shape[0]
  cpw = nch // _NW
  nbuf = 3
  rpt = (n // _NS) // 8 * 8
  rem = n - rpt * _NS

  mesh = plsc.VectorSubcoreMesh(core_axis_name="c", subcore_axis_name="s")

  @functools.partial(
      pl.kernel,
      out_type=jax.ShapeDtypeStruct((_NC, n, dim), jnp.float32),
      mesh=mesh,
      scratch_types=[
          pltpu.VMEM((nbuf, _CH), jnp.int32),
          pltpu.VMEM((nbuf, 1, _CH), jnp.int32),
          pltpu.VMEM((nbuf, _CH), jnp.float32),
          pltpu.VMEM((nbuf, _CH, 128), jnp.float32),
          pltpu.VMEM_SHARED((10000, 128), jnp.float32),
          [pltpu.SemaphoreType.DMA] * nbuf,
          [pltpu.SemaphoreType.DMA] * nbuf,
          [pltpu.SemaphoreType.DMA] * nbuf,
          [pltpu.SemaphoreType.DMA] * nbuf,
      ],
  )
  def body(h_hbm, src_hbm, dst_hbm, w_hbm, out_hbm,
           srcv, dstv, wv, rows_v, acc_sh, isem, dsem, gsem, ssem):
    c = lax.axis_index("c")
    s = lax.axis_index("s")
    wid = s * _NC + c
    first = wid * cpw

    def zrow(i, carry):
      for g in range(dim // _L):
        rows_v[0, i, pl.ds(g * _L, _L)] = jnp.zeros((_L,), jnp.float32)
      return carry

    lax.fori_loop(0, _CH, zrow, 0)
    nfull, tail = divmod(rpt, _CH)
    for k in range(nfull):
      pltpu.sync_copy(
          rows_v.at[0],
          acc_sh.at[pl.ds(s * rpt + k * _CH, _CH)],
      )
    if tail:
      pltpu.sync_copy(
          rows_v.at[0, pl.ds(0, tail)],
          acc_sh.at[pl.ds(s * rpt + nfull * _CH, tail)],
      )
    if rem:
      @pl.when(s == 0)
      def _():
        pltpu.sync_copy(
            rows_v.at[0, pl.ds(0, rem)],
            acc_sh.at[pl.ds(_NS * rpt, rem)],
        )
    plsc.subcore_barrier()

    def issue_src(b, i):
      pltpu.async_copy(src_hbm.at[first + i], srcv.at[b], isem[b])

    def issue_dstw(b, i):
      pltpu.async_copy(dst_hbm.at[first + i], dstv.at[b], dsem[b])
      pltpu.async_copy(w_hbm.at[first + i], wv.at[b], dsem[b])

    def wait_src(b, i):
      pltpu.make_async_copy(src_hbm.at[first + i], srcv.at[b], isem[b]).wait()

    def wait_dstw(b, i):
      pltpu.make_async_copy(dst_hbm.at[first + i], dstv.at[b], dsem[b]).wait()
      pltpu.make_async_copy(w_hbm.at[first + i], wv.at[b], dsem[b]).wait()

    def gissue(b):
      pltpu.async_copy(h_hbm.at[srcv.at[b]], rows_v.at[b], gsem[b])

    def gwait(b):
      pltpu.make_async_copy(h_hbm.at[srcv.at[b]], rows_v.at[b], gsem[b]).wait()

    def sissue(b):
      pltpu.async_copy(rows_v.at[b], acc_sh.at[dstv.at[b, 0]], ssem[b],
                       add=True)

    def swait(b):
      pltpu.make_async_copy(rows_v.at[b], acc_sh.at[dstv.at[b, 0]],
                            ssem[b]).wait()

    def scale(b):
      def grp_body(g, ecarry):
        wgrp = wv[b, pl.ds(g * _L, _L)]
        for j in range(_L):
          ei = g * _L + j
          wb = lax.gather(
              wgrp,
              jnp.full((_L, 1), j, jnp.int32),
              lax.GatherDimensionNumbers(
                  offset_dims=(),
                  collapsed_slice_dims=(0,),
                  start_index_map=(0,),
              ),
              (1,),
              mode=lax.GatherScatterMode.PROMISE_IN_BOUNDS,
          )
          for gg in range(dim // _L):
            rows_v[b, ei, pl.ds(gg * _L, _L)] = (
                rows_v[b, ei, pl.ds(gg * _L, _L)] * wb
            )
        return ecarry

      lax.fori_loop(0, _CH // _L, grp_body, 0)

    issue_src(0, 0)
    issue_dstw(0, 0)
    issue_src(1, 1)
    wait_src(0, 0)
    gissue(0)

    def step(t, carry):
      for k in range(nbuf):
        b = k
        b1 = (k + 1) % nbuf
        b2 = (k + 2) % nbuf
        i = t * nbuf + k

        @pl.when(i >= 2)
        def _():
          swait(b1)

        @pl.when(i + 2 < cpw)
        def _():
          issue_src(b2, i + 2)

        @pl.when(i + 1 < cpw)
        def _():
          issue_dstw(b1, i + 1)
          wait_src(b1, i + 1)
          gissue(b1)

        gwait(b)
        wait_dstw(b, i)
        scale(b)
        sissue(b)
      return carry

    lax.fori_loop(0, cpw // nbuf, step, 0)
    for i in (cpw - 2, cpw - 1):
      swait(i % nbuf)
    plsc.subcore_barrier()

    pltpu.sync_copy(
        acc_sh.at[pl.ds(s * rpt, rpt)],
        out_hbm.at[c, pl.ds(s * rpt, rpt)],
    )
    if rem:
      @pl.when(s == 0)
      def _():
        pltpu.sync_copy(
            acc_sh.at[pl.ds(_NS * rpt, rem)],
            out_hbm.at[c, pl.ds(_NS * rpt, rem)],
        )

  return body(h, src2, dst3, wch)


def _readout(parts, batch3d, W_enc, b_enc, W1, b1, W2, b2):
  n = parts.shape[1]
  dim = parts.shape[2]
  nb = batch3d.shape[0]
  blk = n // nb
  ngraphs = 64
  out_d = W2.shape[1]

  def body(p0_ref, p1_ref, b_ref, we_ref, be_ref, w1_ref, b1_ref, w2_ref,
           b2_ref, o_ref, g_acc):
    i = pl.program_id(0)
    agg = p0_ref[0] + p1_ref[0]
    h2 = jnp.dot(agg, we_ref[...], preferred_element_type=jnp.float32)
    h2 = jnp.maximum(h2 + be_ref[...], 0.0)
    bvec = b_ref[0, 0, :]
    onehot = (
        bvec[None, :]
        == lax.broadcasted_iota(jnp.int32, (ngraphs, blk), 0)
    ).astype(jnp.float32)

    @pl.when(i == 0)
    def _():
      g_acc[...] = jnp.zeros_like(g_acc)

    g_acc[...] += jnp.dot(onehot, h2, preferred_element_type=jnp.float32)

    @pl.when(i == nb - 1)
    def _():
      g = g_acc[...]
      o = jnp.dot(g, w1_ref[...], preferred_element_type=jnp.float32)
      o = jnp.maximum(o + b1_ref[...], 0.0)
      o_ref[...] = (
          jnp.dot(o, w2_ref[...], preferred_element_type=jnp.float32)
          + b2_ref[...]
      )

  return pl.pallas_call(
      body,
      grid=(nb,),
      in_specs=[
          pl.BlockSpec((1, blk, dim), lambda i: (0, i, 0)),
          pl.BlockSpec((1, blk, dim), lambda i: (1, i, 0)),
          pl.BlockSpec((1, 1, blk), lambda i: (i, 0, 0)),
          pl.BlockSpec((dim, dim), lambda i: (0, 0)),
          pl.BlockSpec((1, dim), lambda i: (0, 0)),
          pl.BlockSpec((dim, dim), lambda i: (0, 0)),
          pl.BlockSpec((1, dim), lambda i: (0, 0)),
          pl.BlockSpec((dim, out_d), lambda i: (0, 0)),
          pl.BlockSpec((1, out_d), lambda i: (0, 0)),
      ],
      out_specs=pl.BlockSpec((ngraphs, out_d), lambda i: (0, 0)),
      out_shape=jax.ShapeDtypeStruct((ngraphs, out_d), jnp.float32),
      scratch_shapes=[pltpu.VMEM((ngraphs, dim), jnp.float32)],
  )(parts, parts, batch3d, W_enc, b_enc.reshape(1, -1), W1,
    b1.reshape(1, -1), W2, b2.reshape(1, -1))


def kernel(x, edge_weight, W_pre1, b_pre1, W_pre2, b_pre2, W_enc, b_enc,
           W_post1, b_post1, W_post2, b_post2, edge_index, batch, y):
  h = _pre_mlp(x, W_pre1, b_pre1, W_pre2, b_pre2)
  n = x.shape[0]
  e = edge_weight.shape[0]
  cpw = -(-e // (_NW * _CH))
  cpw = -(-cpw // 3) * 3
  pad = cpw * _NW * _CH - e
  ei32 = edge_index.astype(jnp.int32)
  if pad:
    ar = jnp.arange(pad, dtype=jnp.int32) % n
    edges = jnp.concatenate([ei32, jnp.stack([ar, ar])], axis=1)
    w_all = jnp.concatenate(
        [edge_weight, jnp.zeros((pad,), edge_weight.dtype)]
    )
  else:
    edges, w_all = ei32, edge_weight
  nch = cpw * _NW
  src2 = edges[0].reshape(nch, _CH)
  dst3 = edges[1].reshape(nch, 1, _CH)
  wch = w_all.reshape(nch, _CH)
  parts = _message_passing(h, src2, dst3, wch)
  nb = 10
  batch3d = batch.astype(jnp.int32).reshape(nb, 1, n // nb)
  out = _readout(parts, batch3d, W_enc, b_enc, W_post1, b_post1,
                 W_post2, b_post2)
  return (out, y)


# speedup vs baseline: 11.6662x; 1.0063x over previous
import functools

import jax
import jax.numpy as jnp
from jax import lax
from jax.experimental import pallas as pl
from jax.experimental.pallas import tpu as pltpu
from jax.experimental.pallas import tpu_sc as plsc

_NC = 2
_NS = 16
_L = 16
_NW = _NC * _NS

_CH = 128


def _pre_mlp(x, W1, b1, W2, b2):
  n, d = x.shape
  h = W1.shape[1]
  blk = 1000
  grid = n // blk

  def body(x_ref, w1_ref, b1_ref, w2_ref, b2_ref, o_ref):
    a = jnp.dot(x_ref[...], w1_ref[...], preferred_element_type=jnp.float32)
    a = jnp.maximum(a + b1_ref[...], 0.0)
    o_ref[...] = (
        jnp.dot(a, w2_ref[...], preferred_element_type=jnp.float32)
        + b2_ref[...]
    )

  return pl.pallas_call(
      body,
      grid=(grid,),
      in_specs=[
          pl.BlockSpec((blk, d), lambda i: (i, 0)),
          pl.BlockSpec((d, h), lambda i: (0, 0)),
          pl.BlockSpec((1, h), lambda i: (0, 0)),
          pl.BlockSpec((h, h), lambda i: (0, 0)),
          pl.BlockSpec((1, h), lambda i: (0, 0)),
      ],
      out_specs=pl.BlockSpec((blk, h), lambda i: (i, 0)),
      out_shape=jax.ShapeDtypeStruct((n, h), jnp.float32),
  )(x, W1, b1.reshape(1, -1), W2, b2.reshape(1, -1))


def _message_passing(h, ei, w1d):
  n, dim = h.shape
  nch = ei.shape[1] // _CH
  cpw = nch // _NW
  nbuf = 3
  rpt = (n // _NS) // 8 * 8
  rem = n - rpt * _NS

  mesh = plsc.VectorSubcoreMesh(core_axis_name="c", subcore_axis_name="s")

  @functools.partial(
      pl.kernel,
      out_type=jax.ShapeDtypeStruct((_NC, n, dim), jnp.float32),
      mesh=mesh,
      scratch_types=[
          pltpu.VMEM((nbuf, _CH), jnp.int32),
          pltpu.VMEM((nbuf, _CH), jnp.int32),
          pltpu.VMEM((nbuf, _CH), jnp.float32),
          pltpu.VMEM((nbuf, _CH, 128), jnp.float32),
          pltpu.VMEM_SHARED((10000, 128), jnp.float32),
          [pltpu.SemaphoreType.DMA] * nbuf,
          [pltpu.SemaphoreType.DMA] * nbuf,
          [pltpu.SemaphoreType.DMA] * nbuf,
          [pltpu.SemaphoreType.DMA] * nbuf,
      ],
  )
  def body(h_hbm, ei_hbm, w_hbm, out_hbm,
           srcv, dstv, wv, rows_v, acc_sh, isem, dsem, gsem, ssem):
    c = lax.axis_index("c")
    s = lax.axis_index("s")
    wid = s * _NC + c
    first = wid * cpw

    def zrow(i, carry):
      for g in range(dim // _L):
        rows_v[0, i, pl.ds(g * _L, _L)] = jnp.zeros((_L,), jnp.float32)
      return carry

    lax.fori_loop(0, _CH, zrow, 0)
    nfull, tail = divmod(rpt, _CH)
    for k in range(nfull):
      pltpu.sync_copy(
          rows_v.at[0],
          acc_sh.at[pl.ds(s * rpt + k * _CH, _CH)],
      )
    if tail:
      pltpu.sync_copy(
          rows_v.at[0, pl.ds(0, tail)],
          acc_sh.at[pl.ds(s * rpt + nfull * _CH, tail)],
      )
    if rem:
      @pl.when(s == 0)
      def _():
        pltpu.sync_copy(
            rows_v.at[0, pl.ds(0, rem)],
            acc_sh.at[pl.ds(_NS * rpt, rem)],
        )
    plsc.subcore_barrier()

    def issue_src(b, i):
      base = (first + i) * _CH
      pltpu.async_copy(ei_hbm.at[0, pl.ds(base, _CH)], srcv.at[b], isem[b])

    def issue_dstw(b, i):
      base = (first + i) * _CH
      pltpu.async_copy(ei_hbm.at[1, pl.ds(base, _CH)], dstv.at[b], dsem[b])
      pltpu.async_copy(w_hbm.at[pl.ds(base, _CH)], wv.at[b], dsem[b])

    def wait_src(b, i):
      base = (first + i) * _CH
      pltpu.make_async_copy(
          ei_hbm.at[0, pl.ds(base, _CH)], srcv.at[b], isem[b]
      ).wait()

    def wait_dstw(b, i):
      base = (first + i) * _CH
      pltpu.make_async_copy(
          ei_hbm.at[1, pl.ds(base, _CH)], dstv.at[b], dsem[b]
      ).wait()
      pltpu.make_async_copy(w_hbm.at[pl.ds(base, _CH)], wv.at[b], dsem[b]).wait()

    def gissue(b):
      pltpu.async_copy(h_hbm.at[srcv.at[b]], rows_v.at[b], gsem[b])

    def gwait(b):
      pltpu.make_async_copy(h_hbm.at[srcv.at[b]], rows_v.at[b], gsem[b]).wait()

    def sissue(b):
      pltpu.async_copy(rows_v.at[b], acc_sh.at[dstv.at[b]], ssem[b],
                       add=True)

    def swait(b):
      pltpu.make_async_copy(rows_v.at[b], acc_sh.at[dstv.at[b]],
                            ssem[b]).wait()

    def scale(b):
      @plsc.parallel_loop(0, _CH // _L, 1, unroll=2)
      def grp_body(g):
        wgrp = wv[b, pl.ds(g * _L, _L)]
        for j in range(_L):
          ei = g * _L + j
          wb = lax.gather(
              wgrp,
              jnp.full((_L, 1), j, jnp.int32),
              lax.GatherDimensionNumbers(
                  offset_dims=(),
                  collapsed_slice_dims=(0,),
                  start_index_map=(0,),
              ),
              (1,),
              mode=lax.GatherScatterMode.PROMISE_IN_BOUNDS,
          )
          for gg in range(dim // _L):
            rows_v[b, ei, pl.ds(gg * _L, _L)] = (
                rows_v[b, ei, pl.ds(gg * _L, _L)] * wb
            )

    issue_src(0, 0)
    issue_dstw(0, 0)
    issue_src(1, 1)
    wait_src(0, 0)
    gissue(0)

    def step(t, carry):
      for k in range(nbuf):
        b = k
        b1 = (k + 1) % nbuf
        b2 = (k + 2) % nbuf
        i = t * nbuf + k

        @pl.when(i >= 2)
        def _():
          swait(b1)

        @pl.when(i + 2 < cpw)
        def _():
          issue_src(b2, i + 2)

        @pl.when(i + 1 < cpw)
        def _():
          issue_dstw(b1, i + 1)
          wait_src(b1, i + 1)
          gissue(b1)

        gwait(b)
        wait_dstw(b, i)
        scale(b)
        sissue(b)
      return carry

    lax.fori_loop(0, cpw // nbuf, step, 0)
    for i in (cpw - 2, cpw - 1):
      swait(i % nbuf)
    plsc.subcore_barrier()

    pltpu.sync_copy(
        acc_sh.at[pl.ds(s * rpt, rpt)],
        out_hbm.at[c, pl.ds(s * rpt, rpt)],
    )
    if rem:
      @pl.when(s == 0)
      def _():
        pltpu.sync_copy(
            acc_sh.at[pl.ds(_NS * rpt, rem)],
            out_hbm.at[c, pl.ds(_NS * rpt, rem)],
        )

  return body(h, ei, w1d)


def _readout(parts, batch3d, W_enc, b_enc, W1, b1, W2, b2):
  n = parts.shape[1]
  dim = parts.shape[2]
  nb = batch3d.shape[0]
  blk = n // nb
  ngraphs = 64
  out_d = W2.shape[1]

  def body(p0_ref, p1_ref, b_ref, we_ref, be_ref, w1_ref, b1_ref, w2_ref,
           b2_ref, o_ref, g_acc):
    i = pl.program_id(0)
    agg = p0_ref[0] + p1_ref[0]
    h2 = jnp.dot(agg, we_ref[...], preferred_element_type=jnp.float32)
    h2 = jnp.maximum(h2 + be_ref[...], 0.0)
    bvec = b_ref[0, 0, :]
    onehot = (
        bvec[None, :]
        == lax.broadcasted_iota(jnp.int32, (ngraphs, blk), 0)
    ).astype(jnp.float32)

    @pl.when(i == 0)
    def _():
      g_acc[...] = jnp.zeros_like(g_acc)

    g_acc[...] += jnp.dot(onehot, h2, preferred_element_type=jnp.float32)

    @pl.when(i == nb - 1)
    def _():
      g = g_acc[...]
      o = jnp.dot(g, w1_ref[...], preferred_element_type=jnp.float32)
      o = jnp.maximum(o + b1_ref[...], 0.0)
      o_ref[...] = (
          jnp.dot(o, w2_ref[...], preferred_element_type=jnp.float32)
          + b2_ref[...]
      )

  return pl.pallas_call(
      body,
      grid=(nb,),
      in_specs=[
          pl.BlockSpec((1, blk, dim), lambda i: (0, i, 0)),
          pl.BlockSpec((1, blk, dim), lambda i: (1, i, 0)),
          pl.BlockSpec((1, 1, blk), lambda i: (i, 0, 0)),
          pl.BlockSpec((dim, dim), lambda i: (0, 0)),
          pl.BlockSpec((1, dim), lambda i: (0, 0)),
          pl.BlockSpec((dim, dim), lambda i: (0, 0)),
          pl.BlockSpec((1, dim), lambda i: (0, 0)),
          pl.BlockSpec((dim, out_d), lambda i: (0, 0)),
          pl.BlockSpec((1, out_d), lambda i: (0, 0)),
      ],
      out_specs=pl.BlockSpec((ngraphs, out_d), lambda i: (0, 0)),
      out_shape=jax.ShapeDtypeStruct((ngraphs, out_d), jnp.float32),
      scratch_shapes=[pltpu.VMEM((ngraphs, dim), jnp.float32)],
  )(parts, parts, batch3d, W_enc, b_enc.reshape(1, -1), W1,
    b1.reshape(1, -1), W2, b2.reshape(1, -1))


def kernel(x, edge_weight, W_pre1, b_pre1, W_pre2, b_pre2, W_enc, b_enc,
           W_post1, b_post1, W_post2, b_post2, edge_index, batch, y):
  h = _pre_mlp(x, W_pre1, b_pre1, W_pre2, b_pre2)
  n = x.shape[0]
  e = edge_weight.shape[0]
  cpw = -(-e // (_NW * _CH))
  cpw = -(-cpw // 3) * 3
  pad = cpw * _NW * _CH - e
  ei32 = edge_index.astype(jnp.int32)
  if pad:
    ar = jnp.arange(pad, dtype=jnp.int32) % n
    edges = jnp.concatenate([ei32, jnp.stack([ar, ar])], axis=1)
    w_all = jnp.concatenate(
        [edge_weight, jnp.zeros((pad,), edge_weight.dtype)]
    )
  else:
    edges, w_all = ei32, edge_weight
  parts = _message_passing(h, edges, w_all)
  nb = 10
  batch3d = batch.astype(jnp.int32).reshape(nb, 1, n // nb)
  out = _readout(parts, batch3d, W_enc, b_enc, W_post1, b_post1,
                 W_post2, b_post2)
  return (out, y)
